# baseline (speedup 1.0000x reference)
import jax
import jax.numpy as jnp
from jax import lax
from jax.experimental import pallas as pl
from jax.experimental.pallas import tpu as pltpu
from jax.experimental.pallas import tpu_sc as plsc

BATCH = 16384
NUM_FIELDS = 26
FIELD_SIZE = 100000
TOTAL_ROWS = NUM_FIELDS * FIELD_SIZE
D0 = 16
D1 = 32
BLK = 128
R0B = TOTAL_ROWS * D0 // BLK
R1B = TOTAL_ROWS * D1 // BLK

NUM_WORKERS = 32
SAMPLES_PER_WORKER = BATCH // NUM_WORKERS
CHUNK = 16
NCHUNKS = SAMPLES_PER_WORKER // CHUNK
CIDX = CHUNK * NUM_FIELDS
GSLICE = 128
NGATHER = (CIDX + GSLICE - 1) // GSLICE


def _worker_id():
    return lax.axis_index("s") * 2 + lax.axis_index("c")


def _fm_body(x_hbm, e0_hbm, e1_hbm, l0_hbm, l1_hbm, c_hbm, out_hbm,
             idx_v, b0v, b1v, lo0v, lo1v, rows0, rows1, l0v, l1v,
             cv, out_v, sem):
    wid = _worker_id()

    pltpu.sync_copy(c_hbm, cv)
    w0 = cv[pl.ds(0, 16)]
    w1 = cv[pl.ds(16, 16)]
    b0 = cv[pl.ds(32, 16)]
    b1 = cv[pl.ds(48, 16)]

    iota = lax.iota(jnp.int32, 16)
    base26 = iota * NUM_FIELDS

    wbase = wid * (SAMPLES_PER_WORKER * NUM_FIELDS)

    @pl.loop(0, NCHUNKS)
    def _chunk(c):
        cb = wbase + c * CIDX
        pltpu.sync_copy(x_hbm.at[pl.ds(cb, CIDX)], idx_v)

        for k in range(CIDX // 16):
            fvec = ((iota + (k * 16) % NUM_FIELDS) % NUM_FIELDS) * FIELD_SIZE
            sl = pl.ds(k * 16, 16)
            v = idx_v[sl] + fvec
            idx_v[sl] = v
            b0v[sl] = lax.shift_right_logical(v, 3)
            b1v[sl] = lax.shift_right_logical(v, 2)
            lo0v[sl] = (v & 7) * D0
            lo1v[sl] = (v & 3) * D1

        copies = []
        for k in range(NGATHER):
            n = min(GSLICE, CIDX - k * GSLICE)
            if n == CIDX:
                ib0, ib1, ix, d0, d1, dl0, dl1 = (b0v, b1v, idx_v, rows0,
                                                  rows1, l0v, l1v)
            else:
                sl = pl.ds(k * GSLICE, n)
                ib0, ib1, ix = b0v.at[sl], b1v.at[sl], idx_v.at[sl]
                d0, d1, dl0, dl1 = (rows0.at[sl], rows1.at[sl],
                                    l0v.at[sl], l1v.at[sl])
            copies.append(pltpu.async_copy(e0_hbm.at[ib0], d0, sem))
            copies.append(pltpu.async_copy(e1_hbm.at[ib1], d1, sem))
            copies.append(pltpu.async_copy(l0_hbm.at[ix], dl0, sem))
            copies.append(pltpu.async_copy(l1_hbm.at[ix], dl1, sem))
        for cp in copies:
            cp.wait()

        def lin_f(f, carry):
            a0, a1 = carry
            rowv = base26 + f
            a0 = a0 + plsc.load_gather(l0v, [rowv])
            a1 = a1 + plsc.load_gather(l1v, [rowv])
            return a0, a1

        zf = jnp.zeros((16,), jnp.float32)
        lin0, lin1 = lax.fori_loop(0, NUM_FIELDS, lin_f, (zf, zf))

        def make_ix(rows, lov, ndim):
            def dim_d(d, ix):
                def fld_f(f, carry):
                    a, a2 = carry
                    rowv = base26 + f
                    colv = plsc.load_gather(lov, [rowv]) + d
                    v = plsc.load_gather(rows, [rowv, colv])
                    return a + v, a2 + v * v

                a, a2 = lax.fori_loop(0, NUM_FIELDS, fld_f, (zf, zf))
                return ix + (a * a - a2)

            return lax.fori_loop(0, ndim, dim_d, zf)

        ix0 = make_ix(rows0, lo0v, D0)
        ix1 = make_ix(rows1, lo1v, D1)

        z = w0 * (lin0 + b0 + 0.5 * ix0) + w1 * (lin1 + b1 + 0.5 * ix1)
        sig = 1.0 / (1.0 + jnp.exp(-z))
        out_v[pl.ds(c * CHUNK, 16)] = sig

    pltpu.sync_copy(out_v, out_hbm.at[pl.ds(wid * SAMPLES_PER_WORKER,
                                            SAMPLES_PER_WORKER)])


@jax.jit
def _fm_sc(xf, E0b, E1b, L0, L1, consts):
    mesh = plsc.VectorSubcoreMesh(core_axis_name="c", subcore_axis_name="s")
    return pl.kernel(
        _fm_body,
        out_type=jax.ShapeDtypeStruct((BATCH,), jnp.float32),
        mesh=mesh,
        compiler_params=pltpu.CompilerParams(needs_layout_passes=False,
                                             use_tc_tiling_on_sc=False),
        scratch_types=[
            pltpu.VMEM((CIDX,), jnp.int32),
            pltpu.VMEM((CIDX,), jnp.int32),
            pltpu.VMEM((CIDX,), jnp.int32),
            pltpu.VMEM((CIDX,), jnp.int32),
            pltpu.VMEM((CIDX,), jnp.int32),
            pltpu.VMEM((CIDX, BLK), jnp.float32),
            pltpu.VMEM((CIDX, BLK), jnp.float32),
            pltpu.VMEM((CIDX,), jnp.float32),
            pltpu.VMEM((CIDX,), jnp.float32),
            pltpu.VMEM((64,), jnp.float32),
            pltpu.VMEM((SAMPLES_PER_WORKER,), jnp.float32),
            pltpu.SemaphoreType.DMA,
        ],
    )(xf, E0b, E1b, L0, L1, consts)


def _repack(E, rows_per_call, d):
    n_rows = E.shape[0]
    grid = n_rows // rows_per_call
    gsz = BLK // d
    out_rows = rows_per_call // gsz

    def body(i_ref, o_ref):
        for t in range(gsz):
            o_ref[:, t * d:(t + 1) * d] = i_ref[pl.Slice(t, out_rows, gsz), :]

    return pl.pallas_call(
        body,
        grid=(grid,),
        in_specs=[pl.BlockSpec((rows_per_call, d), lambda g: (g, 0))],
        out_specs=pl.BlockSpec((out_rows, BLK), lambda g: (g, 0)),
        out_shape=jax.ShapeDtypeStruct((n_rows // gsz, BLK), jnp.float32),
    )(E)


def kernel(x, E0, E1, L0, L1, b0, b1, weights):
    xf = x.reshape(-1)
    E0b = _repack(E0, 8000, D0)
    E1b = _repack(E1, 8000, D1)
    consts = jnp.concatenate([
        jnp.broadcast_to(weights[0], (16,)),
        jnp.broadcast_to(weights[1], (16,)),
        jnp.broadcast_to(b0[0], (16,)),
        jnp.broadcast_to(b1[0], (16,)),
    ]).astype(jnp.float32)
    return _fm_sc(xf, E0b, E1b, L0.reshape(-1), L1.reshape(-1), consts)


# speedup vs baseline: 1.3219x; 1.3219x over previous
import functools

import jax
import jax.numpy as jnp
from jax import lax
from jax.experimental import pallas as pl
from jax.experimental.pallas import tpu as pltpu
from jax.experimental.pallas import tpu_sc as plsc

BATCH = 16384
NUM_FIELDS = 26
FIELD_SIZE = 100000
TOTAL_ROWS = NUM_FIELDS * FIELD_SIZE
D0 = 16
D1 = 32

NUM_WORKERS = 32
SAMPLES_PER_WORKER = BATCH // NUM_WORKERS
CHUNK = 32
NCHUNKS = SAMPLES_PER_WORKER // CHUNK
CIDX = CHUNK * NUM_FIELDS
GSLICE = 128
NGATHER = (CIDX + GSLICE - 1) // GSLICE


def _worker_id():
    return lax.axis_index("s") * 2 + lax.axis_index("c")


def _fm_body(x_hbm, e0_hbm, e1_hbm, l0_hbm, l1_hbm, c_hbm, out_hbm,
             idx_v, rows0, rows1, l0v, l1v,
             idx_b, rows0b, rows1b, l0vb, l1vb, cv, out_v, sem, semb):
    wid = _worker_id()

    pltpu.sync_copy(c_hbm, cv)
    w0 = cv[pl.ds(0, 16)]
    w1 = cv[pl.ds(16, 16)]
    b0 = cv[pl.ds(32, 16)]
    b1 = cv[pl.ds(48, 16)]

    iota = lax.iota(jnp.int32, 16)
    base26 = iota * NUM_FIELDS
    zeros16 = iota * 0

    wbase = wid * (SAMPLES_PER_WORKER * NUM_FIELDS)

    def prefetch(cc, idx_v, rows0, rows1, l0v, l1v, sem):
        cb = wbase + cc * CIDX
        pltpu.sync_copy(x_hbm.at[pl.ds(cb, CIDX)], idx_v)
        for k in range(CIDX // 16):
            fvec = ((iota + (k * 16) % NUM_FIELDS) % NUM_FIELDS) * FIELD_SIZE
            sl = pl.ds(k * 16, 16)
            idx_v[sl] = idx_v[sl] + fvec
        for k in range(NGATHER):
            n = min(GSLICE, CIDX - k * GSLICE)
            sl = pl.ds(k * GSLICE, n)
            isl = idx_v.at[sl]
            pltpu.async_copy(e0_hbm.at[isl], rows0.at[sl], sem)
            pltpu.async_copy(e1_hbm.at[isl], rows1.at[sl], sem)
            pltpu.async_copy(l0_hbm.at[isl], l0v.at[sl], sem)
            pltpu.async_copy(l1_hbm.at[isl], l1v.at[sl], sem)

    def drain(idx_v, rows0, rows1, l0v, l1v, sem):
        for k in range(NGATHER):
            n = min(GSLICE, CIDX - k * GSLICE)
            sl = pl.ds(k * GSLICE, n)
            isl = idx_v.at[sl]
            pltpu.make_async_copy(e0_hbm.at[isl], rows0.at[sl], sem).wait()
            pltpu.make_async_copy(e1_hbm.at[isl], rows1.at[sl], sem).wait()
            pltpu.make_async_copy(l0_hbm.at[isl], l0v.at[sl], sem).wait()
            pltpu.make_async_copy(l1_hbm.at[isl], l1v.at[sl], sem).wait()

    def compute(c, rows0, rows1, l0v, l1v):
        for g in range(CHUNK // 16):
            gb = g * 16 * NUM_FIELDS

            def lin_f(f, carry):
                a0, a1 = carry
                rowv = base26 + (gb + f)
                a0 = a0 + plsc.load_gather(l0v, [rowv])
                a1 = a1 + plsc.load_gather(l1v, [rowv])
                return a0, a1

            zf = jnp.zeros((16,), jnp.float32)
            lin0, lin1 = lax.fori_loop(0, NUM_FIELDS, lin_f, (zf, zf))

            def make_ix(rows, ndim):
                def dim_d(d, ix):
                    dvec = jnp.broadcast_to(d, (16,))

                    def fld_f(f, carry):
                        a, a2 = carry
                        rowv = base26 + (gb + f)
                        v = plsc.load_gather(rows, [rowv, dvec])
                        return a + v, a2 + v * v

                    a, a2 = lax.fori_loop(0, NUM_FIELDS, fld_f, (zf, zf))
                    return ix + (a * a - a2)

                return lax.fori_loop(0, ndim, dim_d, zf)

            ix0 = make_ix(rows0, D0)
            ix1 = make_ix(rows1, D1)

            z = w0 * (lin0 + b0 + 0.5 * ix0) + w1 * (lin1 + b1 + 0.5 * ix1)
            sig = 1.0 / (1.0 + jnp.exp(-z))
            out_v[pl.ds(c * CHUNK + g * 16, 16)] = sig

    bufs = ((idx_v, rows0, rows1, l0v, l1v, sem),
            (idx_b, rows0b, rows1b, l0vb, l1vb, semb))
    prefetch(0, *bufs[0])

    @pl.loop(0, NCHUNKS, step=2)
    def _chunk(c):
        for b in range(2):
            cc = c + b
            drain(*bufs[b])

            @pl.when(cc + 1 < NCHUNKS)
            def _():
                prefetch(cc + 1, *bufs[1 - b])

            compute(cc, *bufs[b][1:5])

    pltpu.sync_copy(out_v, out_hbm.at[pl.ds(wid * SAMPLES_PER_WORKER,
                                            SAMPLES_PER_WORKER)])


@jax.jit
def _fm_sc(xf, E0, E1, L0, L1, consts):
    mesh = plsc.VectorSubcoreMesh(core_axis_name="c", subcore_axis_name="s")
    return pl.kernel(
        _fm_body,
        out_type=jax.ShapeDtypeStruct((BATCH,), jnp.float32),
        mesh=mesh,
        compiler_params=pltpu.CompilerParams(needs_layout_passes=False, use_tc_tiling_on_sc=False),
        scratch_types=[
            pltpu.VMEM((CIDX,), jnp.int32),
            pltpu.VMEM((CIDX, D0), jnp.float32),
            pltpu.VMEM((CIDX, D1), jnp.float32),
            pltpu.VMEM((CIDX,), jnp.float32),
            pltpu.VMEM((CIDX,), jnp.float32),
            pltpu.VMEM((CIDX,), jnp.int32),
            pltpu.VMEM((CIDX, D0), jnp.float32),
            pltpu.VMEM((CIDX, D1), jnp.float32),
            pltpu.VMEM((CIDX,), jnp.float32),
            pltpu.VMEM((CIDX,), jnp.float32),
            pltpu.VMEM((64,), jnp.float32),
            pltpu.VMEM((SAMPLES_PER_WORKER,), jnp.float32),
            pltpu.SemaphoreType.DMA,
            pltpu.SemaphoreType.DMA,
        ],
    )(xf, E0, E1, L0, L1, consts)


def kernel(x, E0, E1, L0, L1, b0, b1, weights):
    xf = x.reshape(-1)
    consts = jnp.concatenate([
        jnp.broadcast_to(weights[0], (16,)),
        jnp.broadcast_to(weights[1], (16,)),
        jnp.broadcast_to(b0[0], (16,)),
        jnp.broadcast_to(b1[0], (16,)),
    ]).astype(jnp.float32)
    return _fm_sc(xf, E0, E1, L0.reshape(-1), L1.reshape(-1), consts)
